# Initial kernel scaffold; baseline (speedup 1.0000x reference)
import jax
import jax.numpy as jnp
from jax.experimental import pallas as pl


def kernel(atom_types, t, frac_coords, edge_index, lattices, node2graph, params):
    raise NotImplementedError("write your pallas kernel here")


# speedup vs baseline: 2.4680x; 2.4680x over previous
import functools
import math

import jax
import jax.numpy as jnp
from jax import lax
from jax.experimental import pallas as pl
from jax.experimental.pallas import tpu as pltpu
from jax.experimental.pallas import tpu_sc as plsc

HID = 128
LAT = 256
NFREQ = 10
N_NODES = 10000
N_EDGES = 320000
N_GRAPHS = 256

NCORES = 2
NSUB = 16
NW = NCORES * NSUB
EPW = N_EDGES // NW
CHUNK = 128
NFULL = EPW // CHUNK
TAIL = EPW - NFULL * CHUNK
NPAD = 10240
RPS = NPAD // NSUB

EBLK = 2000
NBLK = 2000

_TWO_PI = 2.0 * math.pi
_F32 = jnp.float32


def _silu(x):
    return x * jax.nn.sigmoid(x)


def _gather_body(with_frac, *refs):
    if with_frac:
        (p_hbm, q_hbm, fp_hbm, src_hbm, dst_hbm,
         hi_hbm, hj_hbm, fs_hbm, fd_hbm,
         idxv, idxt, rowb, rowt, frb, frt, sem) = refs
    else:
        (p_hbm, q_hbm, src_hbm, dst_hbm, hi_hbm, hj_hbm,
         idxv, idxt, rowb, rowt, sem) = refs
        fp_hbm = fs_hbm = fd_hbm = frb = frt = None

    c = lax.axis_index("c")
    s = lax.axis_index("s")
    wid = c * NSUB + s
    base0 = wid * EPW

    def chunk(off, idx_ref, row_ref, fr_ref):
        pltpu.sync_copy(src_hbm.at[pl.ds(off, idx_ref.shape[0])], idx_ref)
        g1 = pltpu.async_copy(p_hbm.at[idx_ref], row_ref, sem)
        if with_frac:
            g2 = pltpu.async_copy(fp_hbm.at[idx_ref], fr_ref, sem)
        g1.wait()
        if with_frac:
            g2.wait()
        pltpu.sync_copy(row_ref, hi_hbm.at[pl.ds(off, row_ref.shape[0])])
        if with_frac:
            pltpu.sync_copy(fr_ref, fs_hbm.at[pl.ds(off, fr_ref.shape[0])])

        pltpu.sync_copy(dst_hbm.at[pl.ds(off, idx_ref.shape[0])], idx_ref)
        g1 = pltpu.async_copy(q_hbm.at[idx_ref], row_ref, sem)
        if with_frac:
            g2 = pltpu.async_copy(fp_hbm.at[idx_ref], fr_ref, sem)
        g1.wait()
        if with_frac:
            g2.wait()
        pltpu.sync_copy(row_ref, hj_hbm.at[pl.ds(off, row_ref.shape[0])])
        if with_frac:
            pltpu.sync_copy(fr_ref, fd_hbm.at[pl.ds(off, fr_ref.shape[0])])

    def body(j, carry):
        chunk(base0 + j * CHUNK, idxv, rowb, frb)
        return carry

    lax.fori_loop(0, NFULL, body, 0)
    chunk(base0 + NFULL * CHUNK, idxt, rowt, frt)


def _make_gather(with_frac):
    nout = 4 if with_frac else 2
    outs = [jax.ShapeDtypeStruct((N_EDGES, HID), _F32)] * nout
    scratch = [pltpu.VMEM((CHUNK,), jnp.int32),
               pltpu.VMEM((TAIL,), jnp.int32),
               pltpu.VMEM((CHUNK, HID), _F32),
               pltpu.VMEM((TAIL, HID), _F32)]
    if with_frac:
        scratch += [pltpu.VMEM((CHUNK, HID), _F32),
                    pltpu.VMEM((TAIL, HID), _F32)]
    scratch += [pltpu.SemaphoreType.DMA]
    mesh = plsc.VectorSubcoreMesh(core_axis_name="c", subcore_axis_name="s")
    return pl.kernel(functools.partial(_gather_body, with_frac),
                     out_type=tuple(outs), mesh=mesh,
                     scratch_types=tuple(scratch))


def _scatter_body(with_vals, *refs):
    if with_vals:
        (val_hbm, didx_hbm, zeros_hbm, out_hbm,
         idxv, idxt, valb, valt, shacc) = refs
    else:
        (didx_hbm, zeros_hbm, ones_hbm, out_hbm,
         idxv, idxt, valb, valt, shacc) = refs

    c = lax.axis_index("c")
    s = lax.axis_index("s")
    wid = c * NSUB + s
    base0 = wid * EPW
    rbase = s * RPS

    pltpu.sync_copy(zeros_hbm.at[pl.ds(rbase, RPS)], shacc.at[pl.ds(rbase, RPS)])
    if not with_vals:
        pltpu.sync_copy(ones_hbm, valb)
        pltpu.sync_copy(ones_hbm.at[pl.ds(0, TAIL)], valt)
    plsc.subcore_barrier()

    def chunk(off, idx_ref, val_ref):
        pltpu.sync_copy(didx_hbm.at[pl.ds(off, idx_ref.shape[0])], idx_ref)
        if with_vals:
            pltpu.sync_copy(val_hbm.at[pl.ds(off, val_ref.shape[0])], val_ref)
        pltpu.sync_copy(val_ref, shacc.at[idx_ref], add=True)

    def body(j, carry):
        chunk(base0 + j * CHUNK, idxv, valb)
        return carry

    lax.fori_loop(0, NFULL, body, 0)
    chunk(base0 + NFULL * CHUNK, idxt, valt)
    plsc.subcore_barrier()

    pltpu.sync_copy(shacc.at[pl.ds(rbase, RPS)],
                    out_hbm.at[c, pl.ds(rbase, RPS)])


def _make_scatter(with_vals):
    outs = [jax.ShapeDtypeStruct((2, NPAD, HID), _F32)]
    scratch = [pltpu.VMEM((CHUNK,), jnp.int32),
               pltpu.VMEM((TAIL,), jnp.int32),
               pltpu.VMEM((CHUNK, HID), _F32),
               pltpu.VMEM((TAIL, HID), _F32),
               pltpu.VMEM_SHARED((NPAD, HID), _F32)]
    mesh = plsc.VectorSubcoreMesh(core_axis_name="c", subcore_axis_name="s")
    return pl.kernel(functools.partial(_scatter_body, with_vals),
                     out_type=tuple(outs), mesh=mesh,
                     scratch_types=tuple(scratch))


def _full(shape):
    return pl.BlockSpec(shape, lambda i: (0,) * len(shape))


def _init_body(at_ref, t_ref, n2g_ref, latips_ref,
               nembW_ref, latWs_ref, latWc_ref, latb_ref,
               w1s_ref, w1d_ref, w1lat_ref, b1_ref,
               nf_ref, ln_ref, p_ref, q_ref):
    at = at_ref[...]
    blk = at.shape[0]
    oh_atom = (at - 1 == lax.broadcasted_iota(jnp.int32, (blk, HID), 1)
               ).astype(_F32)
    n2g = n2g_ref[...]
    oh_g = (n2g == lax.broadcasted_iota(jnp.int32, (blk, N_GRAPHS), 1)
            ).astype(_F32)
    ln = jnp.dot(oh_g, latips_ref[...], preferred_element_type=_F32)
    fac = math.log(10000.0) / (LAT // 2 - 1)
    lane = lax.broadcasted_iota(jnp.int32, (blk, HID), 1).astype(_F32)
    ang = t_ref[...] * jnp.exp(lane * (-fac))
    nf = (jnp.dot(oh_atom, nembW_ref[...], preferred_element_type=_F32)
          + jnp.dot(jnp.sin(ang), latWs_ref[...], preferred_element_type=_F32)
          + jnp.dot(jnp.cos(ang), latWc_ref[...], preferred_element_type=_F32)
          + latb_ref[...])
    nf_ref[...] = nf
    ln_ref[...] = ln
    p_ref[...] = (jnp.dot(nf, w1s_ref[...], preferred_element_type=_F32)
                  + jnp.dot(ln, w1lat_ref[...], preferred_element_type=_F32)
                  + b1_ref[...])
    q_ref[...] = jnp.dot(nf, w1d_ref[...], preferred_element_type=_F32)


def _fd_emb_matmul(fd, ws_ref, wc_ref):
    blk = fd.shape[0]
    j = lax.broadcasted_iota(jnp.int32, (blk, 32), 1)
    k = (j % NFREQ).astype(_F32)
    xs = jnp.where(j < NFREQ, fd[:, 0:1],
                   jnp.where(j < 2 * NFREQ, fd[:, 1:2], fd[:, 2:3]))
    ang = (_TWO_PI * k) * xs
    return (jnp.dot(jnp.sin(ang), ws_ref[...], preferred_element_type=_F32)
            + jnp.dot(jnp.cos(ang), wc_ref[...], preferred_element_type=_F32))


def _edge0_body(hi_ref, hj_ref, fs_ref, fdt_ref, ws_ref, wc_ref, w2_ref,
                b2_ref, ef_ref, fd16_ref):
    shift = fdt_ref[...][:, 0:16] - fs_ref[...][:, 0:16]
    fd = shift - jnp.floor(shift + 0.5 + 1e-4)
    fd16_ref[...] = fd
    x = hi_ref[...] + hj_ref[...] + _fd_emb_matmul(fd, ws_ref, wc_ref)
    h = _silu(x)
    ef_ref[...] = _silu(jnp.dot(h, w2_ref[...], preferred_element_type=_F32)
                        + b2_ref[...])


def _edge_body(hi_ref, hj_ref, fd16_ref, ws_ref, wc_ref, w2_ref, b2_ref,
               ef_ref):
    x = (hi_ref[...] + hj_ref[...]
         + _fd_emb_matmul(fd16_ref[...], ws_ref, wc_ref))
    h = _silu(x)
    ef_ref[...] = _silu(jnp.dot(h, w2_ref[...], preferred_element_type=_F32)
                        + b2_ref[...])


def _node_body(nf_ref, ln_ref, a0_ref, a1_ref, c0_ref, c1_ref,
               nW1a_ref, nW1b_ref, nb1_ref, nW2_ref, nb2_ref,
               w1s_ref, w1d_ref, w1lat_ref, b1_ref,
               nfo_ref, p_ref, q_ref):
    cnt = c0_ref[...][:, 0:1] + c1_ref[...][:, 0:1]
    recip = 1.0 / jnp.maximum(cnt, 1.0)
    agg = (a0_ref[...] + a1_ref[...]) * recip
    nf = nf_ref[...]
    h = _silu(jnp.dot(nf, nW1a_ref[...], preferred_element_type=_F32)
              + jnp.dot(agg, nW1b_ref[...], preferred_element_type=_F32)
              + nb1_ref[...])
    nfo = nf + _silu(jnp.dot(h, nW2_ref[...], preferred_element_type=_F32)
                     + nb2_ref[...])
    nfo_ref[...] = nfo
    ln = ln_ref[...]
    p_ref[...] = (jnp.dot(nfo, w1s_ref[...], preferred_element_type=_F32)
                  + jnp.dot(ln, w1lat_ref[...], preferred_element_type=_F32)
                  + b1_ref[...])
    q_ref[...] = jnp.dot(nfo, w1d_ref[...], preferred_element_type=_F32)


def _final_body(ef_ref, fd16_ref, cW1_ref, cb1_ref, cW2_ref, tr_ref):
    blk = ef_ref.shape[0]
    fd = fd16_ref[...]
    h = _silu(jnp.dot(ef_ref[...], cW1_ref[...], preferred_element_type=_F32)
              + cb1_ref[...])
    sc = jnp.dot(h, cW2_ref[...], preferred_element_type=_F32)
    tr_ref[...] = jnp.concatenate(
        [fd * sc, jnp.zeros((blk, HID - 16), _F32)], axis=1)


def _out_body(t0_ref, t1_ref, c0_ref, c1_ref, o_ref):
    cnt = c0_ref[...][:, 0:1] + c1_ref[...][:, 0:1]
    recip = 1.0 / jnp.maximum(cnt, 1.0)
    o_ref[...] = (t0_ref[...][:, 0:3] + t1_ref[...][:, 0:3]) * recip


def kernel(atom_types, t, frac_coords, edge_index, lattices, node2graph,
           params):
    n = N_NODES
    e = N_EDGES
    src = edge_index[0].astype(jnp.int32)
    dst = edge_index[1].astype(jnp.int32)
    at2 = atom_types.astype(jnp.int32).reshape(n, 1)
    t2 = t.reshape(n, 1)
    n2g2 = node2graph.astype(jnp.int32).reshape(n, 1)
    fracp = jnp.pad(frac_coords, ((0, 0), (0, HID - 3)))
    lat_ips = (lattices @ jnp.swapaxes(lattices, -1, -2)).reshape(-1, 9)
    latips16 = jnp.pad(lat_ips, ((0, N_GRAPHS - lat_ips.shape[0]), (0, 7)))

    nemb_pad = jnp.pad(params["node_emb"],
                       ((0, HID - params["node_emb"].shape[0]), (0, 0)))
    latW = params["lat_W"]
    nembW = nemb_pad @ latW[:HID]
    latWs = latW[HID:HID + LAT // 2]
    latWc = latW[HID + LAT // 2:]
    latb = params["lat_b"].reshape(1, HID)
    lw = []
    for lp in params["layers"]:
        W1 = lp["e_W1"]
        lw.append(dict(
            w1s=W1[:HID], w1d=W1[HID:2 * HID],
            w1lat=jnp.pad(W1[2 * HID:2 * HID + 9], ((0, 7), (0, 0))),
            b1=lp["e_b1"].reshape(1, HID),
            ws=jnp.pad(W1[2 * HID + 9:2 * HID + 9 + 30], ((0, 2), (0, 0))),
            wc=jnp.pad(W1[2 * HID + 39:], ((0, 2), (0, 0))),
            w2=lp["e_W2"], b2=lp["e_b2"].reshape(1, HID),
            nW1a=lp["n_W1"][:HID], nW1b=lp["n_W1"][HID:],
            nb1=lp["n_b1"].reshape(1, HID),
            nW2=lp["n_W2"], nb2=lp["n_b2"].reshape(1, HID),
        ))
    zerosN = jnp.zeros((NPAD, HID), _F32)
    ones128 = jnp.ones((CHUNK, HID), _F32)

    ng = n // NBLK
    nodeblk = lambda w: pl.BlockSpec((NBLK, w), lambda i: (i, 0))
    edgeblk = lambda w: pl.BlockSpec((EBLK, w), lambda i: (i, 0))
    wspec = [_full((HID, HID)), _full((HID, HID)), _full((16, HID)),
             _full((1, HID))]

    init_call = pl.pallas_call(
        _init_body,
        grid=(ng,),
        in_specs=[nodeblk(1), nodeblk(1), nodeblk(1),
                  _full((N_GRAPHS, 16)),
                  _full((HID, HID)), _full((HID, HID)), _full((HID, HID)),
                  _full((1, HID))] + wspec,
        out_specs=[nodeblk(HID), nodeblk(16), nodeblk(HID), nodeblk(HID)],
        out_shape=[jax.ShapeDtypeStruct((n, HID), _F32),
                   jax.ShapeDtypeStruct((n, 16), _F32),
                   jax.ShapeDtypeStruct((n, HID), _F32),
                   jax.ShapeDtypeStruct((n, HID), _F32)],
    )
    l0 = lw[0]
    nf, ln16, p, q = init_call(at2, t2, n2g2, latips16, nembW, latWs, latWc,
                               latb, l0["w1s"], l0["w1d"], l0["w1lat"],
                               l0["b1"])

    eg = e // EBLK
    edge_wspec = [_full((32, HID)), _full((32, HID)), _full((HID, HID)),
                  _full((1, HID))]
    edge0_call = pl.pallas_call(
        _edge0_body,
        grid=(eg,),
        in_specs=[edgeblk(HID)] * 4 + edge_wspec,
        out_specs=[edgeblk(HID), edgeblk(16)],
        out_shape=[jax.ShapeDtypeStruct((e, HID), _F32),
                   jax.ShapeDtypeStruct((e, 16), _F32)],
    )
    edge_call = pl.pallas_call(
        _edge_body,
        grid=(eg,),
        in_specs=[edgeblk(HID), edgeblk(HID), edgeblk(16)] + edge_wspec,
        out_specs=edgeblk(HID),
        out_shape=jax.ShapeDtypeStruct((e, HID), _F32),
    )

    node_call = pl.pallas_call(
        _node_body,
        grid=(ng,),
        in_specs=[nodeblk(HID), nodeblk(16), nodeblk(HID), nodeblk(HID),
                  nodeblk(HID), nodeblk(HID),
                  _full((HID, HID)), _full((HID, HID)), _full((1, HID)),
                  _full((HID, HID)), _full((1, HID))] + wspec,
        out_specs=[nodeblk(HID), nodeblk(HID), nodeblk(HID)],
        out_shape=[jax.ShapeDtypeStruct((n, HID), _F32)] * 3,
    )

    gather0 = _make_gather(True)
    gatherL = _make_gather(False)
    scatterV = _make_scatter(True)
    scatterC = _make_scatter(False)

    (cntp,) = scatterC(dst, zerosN, ones128)
    cp0, cp1 = cntp[0], cntp[1]

    fd16 = None
    ef = None
    for li in range(4):
        w = lw[li]
        if li == 0:
            hi, hj, fs, fdt = gather0(p, q, fracp, src, dst)
            ef, fd16 = edge0_call(hi, hj, fs, fdt, w["ws"], w["wc"],
                                  w["w2"], w["b2"])
        else:
            hi, hj = gatherL(p, q, src, dst)
            ef = edge_call(hi, hj, fd16, w["ws"], w["wc"], w["w2"], w["b2"])
        if li == 3:
            break
        (aggp,) = scatterV(ef, dst, zerosN)
        wn = lw[li + 1]
        nf, p, q = node_call(nf, ln16, aggp[0], aggp[1], cp0, cp1,
                             w["nW1a"], w["nW1b"], w["nb1"], w["nW2"],
                             w["nb2"], wn["w1s"], wn["w1d"], wn["w1lat"],
                             wn["b1"])

    final_call = pl.pallas_call(
        _final_body,
        grid=(eg,),
        in_specs=[edgeblk(HID), edgeblk(16),
                  _full((HID, HID)), _full((1, HID)), _full((HID, 1))],
        out_specs=edgeblk(HID),
        out_shape=jax.ShapeDtypeStruct((e, HID), _F32),
    )
    trans = final_call(ef, fd16, params["c_W1"],
                       params["c_b1"].reshape(1, HID), params["c_W2"])

    (transp,) = scatterV(trans, dst, zerosN)

    out_call = pl.pallas_call(
        _out_body,
        grid=(1,),
        in_specs=[_full((n, HID))] * 4,
        out_specs=_full((n, 3)),
        out_shape=jax.ShapeDtypeStruct((n, 3), _F32),
    )
    return out_call(transp[0], transp[1], cp0, cp1)
